# Initial kernel scaffold; baseline (speedup 1.0000x reference)
import jax
import jax.numpy as jnp
from jax.experimental import pallas as pl


def kernel(x, edge_index, W_in, b_in, W1, b1, W2, b2, W3, b3, gamma, beta, W_out, b_out):
    raise NotImplementedError("write your pallas kernel here")


# speedup vs baseline: 6.0715x; 6.0715x over previous
import functools

import jax
import jax.numpy as jnp
from jax import lax
from jax.experimental import pallas as pl
from jax.experimental.pallas import tpu as pltpu
from jax.experimental.pallas import tpu_sc as plsc

N = 10000
E = 320000
F = 128
NC = 2
NS = 16
CH = 80

EPT_AGG = E // NS
NCH_AGG = EPT_AGG // CH
EPT_DEG = E // (NC * NS)
NCH_DEG = EPT_DEG // CH
STRIPE = 640
LAST = N - 15 * STRIPE

BR = 2000
GRID = N // BR


@functools.cache
def _mesh():
    return plsc.VectorSubcoreMesh(
        core_axis_name="c", subcore_axis_name="s", num_cores=NC,
        num_subcores=NS,
    )


def _stripe_copy(src_at, dst_at, s):
    @pl.when(s < NS - 1)
    def _():
        pltpu.sync_copy(src_at(s * STRIPE, STRIPE), dst_at(s * STRIPE, STRIPE))

    @pl.when(s == NS - 1)
    def _():
        pltpu.sync_copy(src_at(15 * STRIPE, LAST), dst_at(15 * STRIPE, LAST))


def _agg_body(g_hbm, src2_hbm, dst_hbm, out_hbm, isrc_v, idst_v, rows_v,
              acc_sh, sem):
    c = lax.axis_index("c")
    s = lax.axis_index("s")
    _stripe_copy(lambda o, n: g_hbm.at[pl.ds(c * N + o, n)],
                 lambda o, n: acc_sh.at[pl.ds(o, n)], s)
    plsc.subcore_barrier()

    def chunk(k, _):
        base = pl.multiple_of(s * EPT_AGG + k * CH, CH)
        pltpu.sync_copy(src2_hbm.at[pl.ds(c * E + base, CH)], isrc_v)
        pltpu.sync_copy(dst_hbm.at[pl.ds(base, CH)], idst_v)
        pltpu.async_copy(g_hbm.at[isrc_v], rows_v, sem).wait()
        pltpu.sync_copy(rows_v, acc_sh.at[idst_v], add=True)
        return 0

    lax.fori_loop(0, NCH_AGG, chunk, 0)
    plsc.subcore_barrier()
    _stripe_copy(lambda o, n: acc_sh.at[pl.ds(o, n)],
                 lambda o, n: out_hbm.at[pl.ds(c * N + o, n)], s)


@functools.cache
def _agg_kernel():
    return pl.kernel(
        _agg_body,
        out_type=jax.ShapeDtypeStruct((NC * N, F), jnp.float32),
        mesh=_mesh(),
        scratch_types=[
            pltpu.VMEM((CH,), jnp.int32),
            pltpu.VMEM((CH,), jnp.int32),
            pltpu.VMEM((CH, F), jnp.float32),
            pltpu.VMEM_SHARED((N, F), jnp.float32),
            pltpu.SemaphoreType.DMA,
        ],
    )


def _agg_call(g_flat, src2, dst):
    return _agg_kernel()(g_flat, src2, dst)


def _mm(a, b):
    return jnp.dot(a, b, precision=lax.Precision.HIGHEST,
                   preferred_element_type=jnp.float32)


def _gelu(v):
    return 0.5 * v * (1.0 + lax.erf(v * 0.7071067811865476))


def _dinv(degp_ref):
    return lax.rsqrt(degp_ref[:, 0:1])


def _full(shape):
    return pl.BlockSpec(shape, lambda i: (0,) * len(shape))


def _tc_pre_body(x_ref, wi_ref, bi_ref, w1_ref, degp_ref, g_ref):
    dinv = _dinv(degp_ref)
    x = x_ref[...]
    for j in range(2):
        h_j = dinv * _gelu(_mm(x, wi_ref[:, j * F:(j + 1) * F])
                           + bi_ref[0, pl.ds(j * F, F)][None, :])
        for q in range(2):
            blk = _mm(h_j, w1_ref[pl.ds(j * F, F), pl.ds(q * F, F)])
            if j == 0:
                g_ref[q] = blk
            else:
                g_ref[q] += blk


def _tc_pre(x, W_in, b_in, W1, degp):
    return pl.pallas_call(
        _tc_pre_body,
        grid=(GRID,),
        in_specs=[
            pl.BlockSpec((BR, 128), lambda i: (i, 0)),
            _full((128, 256)),
            _full((1, 256)),
            _full((256, 256)),
            pl.BlockSpec((BR, 128), lambda i: (i, 0)),
        ],
        out_specs=pl.BlockSpec((NC, BR, F), lambda i: (0, i, 0)),
        out_shape=jax.ShapeDtypeStruct((NC, N, F), jnp.float32),
    )(x, W_in, b_in.reshape(1, 256), W1, degp)


def _pre_half(a_ref, dinv, b_ref, j):
    return dinv * a_ref[j] + b_ref[0, pl.ds(j * F, F)][None, :]


def _tc_stats_body(a_ref, degp_ref, b_ref, stat_ref):
    i = pl.program_id(0)
    dinv = _dinv(degp_ref)
    for j in range(2):
        pre = _pre_half(a_ref, dinv, b_ref, j)
        sm = jnp.sum(pre, axis=0, keepdims=True)
        sq = jnp.sum(pre * pre, axis=0, keepdims=True)

        @pl.when(i == 0)
        def _():
            stat_ref[0:1, pl.ds(j * F, F)] = sm
            stat_ref[1:2, pl.ds(j * F, F)] = sq

        @pl.when(i > 0)
        def _():
            stat_ref[0:1, pl.ds(j * F, F)] += sm
            stat_ref[1:2, pl.ds(j * F, F)] += sq


def _tc_stats(a, degp, b):
    return pl.pallas_call(
        _tc_stats_body,
        grid=(GRID,),
        in_specs=[
            pl.BlockSpec((NC, BR, F), lambda i: (0, i, 0)),
            pl.BlockSpec((BR, 128), lambda i: (i, 0)),
            _full((1, 256)),
        ],
        out_specs=_full((8, 256)),
        out_shape=jax.ShapeDtypeStruct((8, 256), jnp.float32),
    )(a, degp, b.reshape(1, 256))


def _norm_half(a_ref, b_ref, stat_ref, gamma_ref, beta_ref, j, dinv):
    pre = _pre_half(a_ref, dinv, b_ref, j)
    m = stat_ref[0:1, pl.ds(j * F, F)] * (1.0 / N)
    sq = stat_ref[1:2, pl.ds(j * F, F)] * (1.0 / N)
    v = sq - m * m
    return _gelu((pre - m) * lax.rsqrt(v + 1e-5)
                 * gamma_ref[0, pl.ds(j * F, F)][None, :]
                 + beta_ref[0, pl.ds(j * F, F)][None, :])


def _tc_mid_body(a_ref, degp_ref, b_ref, stat_ref, gamma_ref, beta_ref,
                 w_ref, g_ref):
    dinv = _dinv(degp_ref)
    for j in range(2):
        h_j = dinv * _norm_half(a_ref, b_ref, stat_ref, gamma_ref, beta_ref,
                                j, dinv)
        for q in range(2):
            blk = _mm(h_j, w_ref[pl.ds(j * F, F), pl.ds(q * F, F)])
            if j == 0:
                g_ref[q] = blk
            else:
                g_ref[q] += blk


def _tc_mid(a, degp, b, stat, gamma, beta, W):
    return pl.pallas_call(
        _tc_mid_body,
        grid=(GRID,),
        in_specs=[
            pl.BlockSpec((NC, BR, F), lambda i: (0, i, 0)),
            pl.BlockSpec((BR, 128), lambda i: (i, 0)),
            _full((1, 256)),
            _full((8, 256)),
            _full((1, 256)),
            _full((1, 256)),
            _full((256, 256)),
        ],
        out_specs=pl.BlockSpec((NC, BR, F), lambda i: (0, i, 0)),
        out_shape=jax.ShapeDtypeStruct((NC, N, F), jnp.float32),
    )(a, degp, b.reshape(1, 256), stat, gamma.reshape(1, 256),
      beta.reshape(1, 256), W)


def _tc_final_body(a_ref, degp_ref, b_ref, stat_ref, gamma_ref, beta_ref,
                   wo_ref, bo_ref, out_ref):
    dinv = _dinv(degp_ref)
    for j in range(2):
        h_j = _norm_half(a_ref, b_ref, stat_ref, gamma_ref, beta_ref, j,
                         dinv)
        blk = _mm(h_j, wo_ref[pl.ds(j * F, F), :])
        if j == 0:
            out_ref[...] = blk + bo_ref[...]
        else:
            out_ref[...] += blk


def _tc_final(a, degp, b, stat, gamma, beta, W_out, b_out):
    nout = W_out.shape[1]
    return pl.pallas_call(
        _tc_final_body,
        grid=(GRID,),
        in_specs=[
            pl.BlockSpec((NC, BR, F), lambda i: (0, i, 0)),
            pl.BlockSpec((BR, 128), lambda i: (i, 0)),
            _full((1, 256)),
            _full((8, 256)),
            _full((1, 256)),
            _full((1, 256)),
            _full((256, nout)),
            _full((1, nout)),
        ],
        out_specs=pl.BlockSpec((BR, nout), lambda i: (i, 0)),
        out_shape=jax.ShapeDtypeStruct((N, nout), jnp.float32),
    )(a, degp, b.reshape(1, 256), stat, gamma.reshape(1, 256),
      beta.reshape(1, 256), W_out, b_out.reshape(1, nout))


def kernel(x, edge_index, W_in, b_in, W1, b1, W2, b2, W3, b3, gamma, beta,
           W_out, b_out):
    src = edge_index[0]
    dst = edge_index[1]
    src2 = jnp.concatenate([src, src + jnp.int32(N)])

    ones_flat = jnp.ones((NC * N, F), jnp.float32)
    degp = _agg_call(ones_flat, src2, dst)

    g1 = _tc_pre(x, W_in, b_in, W1, degp)
    a1 = _agg_call(g1.reshape(NC * N, F), src2, dst).reshape(NC, N, F)
    s1 = _tc_stats(a1, degp, b1)
    g2 = _tc_mid(a1, degp, b1, s1, gamma, beta, W2)
    a2 = _agg_call(g2.reshape(NC * N, F), src2, dst).reshape(NC, N, F)
    s2 = _tc_stats(a2, degp, b2)
    g3 = _tc_mid(a2, degp, b2, s2, gamma, beta, W3)
    a3 = _agg_call(g3.reshape(NC * N, F), src2, dst).reshape(NC, N, F)
    s3 = _tc_stats(a3, degp, b3)
    return _tc_final(a3, degp, b3, s3, gamma, beta, W_out, b_out)


# speedup vs baseline: 14.4139x; 2.3740x over previous
import functools

import jax
import jax.numpy as jnp
from jax import lax
from jax.experimental import pallas as pl
from jax.experimental.pallas import tpu as pltpu
from jax.experimental.pallas import tpu_sc as plsc

N = 10000
E = 320000
F = 128
NC = 2
NS = 16
CH = 80

EPT_AGG = E // NS
NCH_AGG = EPT_AGG // CH
EPT_DEG = E // (NC * NS)
NCH_DEG = EPT_DEG // CH
STRIPE = 640
LAST = N - 15 * STRIPE

BR = 2000
GRID = N // BR


@functools.cache
def _mesh():
    return plsc.VectorSubcoreMesh(
        core_axis_name="c", subcore_axis_name="s", num_cores=NC,
        num_subcores=NS,
    )


def _stripe_copy(src_at, dst_at, s):
    @pl.when(s < NS - 1)
    def _():
        pltpu.sync_copy(src_at(s * STRIPE, STRIPE), dst_at(s * STRIPE, STRIPE))

    @pl.when(s == NS - 1)
    def _():
        pltpu.sync_copy(src_at(15 * STRIPE, LAST), dst_at(15 * STRIPE, LAST))


IBLK = 4000
NB_BLK = EPT_AGG // IBLK
CH_BLK = IBLK // CH
STEPS = CH_BLK // 2


def _agg_body(g_hbm, src2_hbm, dst_hbm, out_hbm, iall_s, iall_d, isrc_a,
              idst_a, isrc_b, idst_b, rows_a, rows_b, acc_sh, sem_a, sem_b):
    c = lax.axis_index("c")
    s = lax.axis_index("s")
    _stripe_copy(lambda o, n: g_hbm.at[pl.ds(c * N + o, n)],
                 lambda o, n: acc_sh.at[pl.ds(o, n)], s)
    plsc.subcore_barrier()

    def fill(iv_s, iv_d, k):
        for m in range(CH // 16):
            sl = pl.ds(pl.multiple_of(k * CH + m * 16, 16), 16)
            iv_s[pl.ds(m * 16, 16)] = iall_s[sl]
            iv_d[pl.ds(m * 16, 16)] = iall_d[sl]

    def block(b, _):
        bbase = pl.multiple_of(s * EPT_AGG + b * IBLK, CH)
        pltpu.sync_copy(src2_hbm.at[pl.ds(c * E + bbase, IBLK)], iall_s)
        pltpu.sync_copy(dst_hbm.at[pl.ds(bbase, IBLK)], iall_d)
        fill(isrc_a, idst_a, 0)
        pltpu.async_copy(g_hbm.at[isrc_a], rows_a, sem_a)

        def step(t, _):
            e = 2 * t
            fill(isrc_b, idst_b, e + 1)
            pltpu.async_copy(g_hbm.at[isrc_b], rows_b, sem_b)
            pltpu.make_async_copy(g_hbm.at[isrc_a], rows_a, sem_a).wait()
            pltpu.sync_copy(rows_a, acc_sh.at[idst_a], add=True)

            @pl.when(t < STEPS - 1)
            def _():
                fill(isrc_a, idst_a, e + 2)
                pltpu.async_copy(g_hbm.at[isrc_a], rows_a, sem_a)

            pltpu.make_async_copy(g_hbm.at[isrc_b], rows_b, sem_b).wait()
            pltpu.sync_copy(rows_b, acc_sh.at[idst_b], add=True)
            return 0

        lax.fori_loop(0, STEPS, step, 0)
        return 0

    lax.fori_loop(0, NB_BLK, block, 0)
    plsc.subcore_barrier()
    _stripe_copy(lambda o, n: acc_sh.at[pl.ds(o, n)],
                 lambda o, n: out_hbm.at[pl.ds(c * N + o, n)], s)


@functools.cache
def _agg_kernel():
    return pl.kernel(
        _agg_body,
        out_type=jax.ShapeDtypeStruct((NC * N, F), jnp.float32),
        mesh=_mesh(),
        scratch_types=[
            pltpu.VMEM((IBLK,), jnp.int32),
            pltpu.VMEM((IBLK,), jnp.int32),
            pltpu.VMEM((CH,), jnp.int32),
            pltpu.VMEM((CH,), jnp.int32),
            pltpu.VMEM((CH,), jnp.int32),
            pltpu.VMEM((CH,), jnp.int32),
            pltpu.VMEM((CH, F), jnp.float32),
            pltpu.VMEM((CH, F), jnp.float32),
            pltpu.VMEM_SHARED((N, F), jnp.float32),
            pltpu.SemaphoreType.DMA,
            pltpu.SemaphoreType.DMA,
        ],
    )


def _agg_call(g_flat, src2, dst):
    return _agg_kernel()(g_flat, src2, dst)


def _mm(a, b):
    return jnp.dot(a, b, precision=lax.Precision.HIGHEST,
                   preferred_element_type=jnp.float32)


def _gelu(v):
    return 0.5 * v * (1.0 + lax.erf(v * 0.7071067811865476))


def _dinv(degp_ref):
    return lax.rsqrt(degp_ref[:, 0:1])


def _full(shape):
    return pl.BlockSpec(shape, lambda i: (0,) * len(shape))


def _tc_pre_body(x_ref, wi_ref, bi_ref, w1_ref, degp_ref, g_ref):
    dinv = _dinv(degp_ref)
    x = x_ref[...]
    for j in range(2):
        h_j = dinv * _gelu(_mm(x, wi_ref[:, j * F:(j + 1) * F])
                           + bi_ref[0, pl.ds(j * F, F)][None, :])
        for q in range(2):
            blk = _mm(h_j, w1_ref[pl.ds(j * F, F), pl.ds(q * F, F)])
            if j == 0:
                g_ref[q] = blk
            else:
                g_ref[q] += blk


def _tc_pre(x, W_in, b_in, W1, degp):
    return pl.pallas_call(
        _tc_pre_body,
        grid=(GRID,),
        in_specs=[
            pl.BlockSpec((BR, 128), lambda i: (i, 0)),
            _full((128, 256)),
            _full((1, 256)),
            _full((256, 256)),
            pl.BlockSpec((BR, 128), lambda i: (i, 0)),
        ],
        out_specs=pl.BlockSpec((NC, BR, F), lambda i: (0, i, 0)),
        out_shape=jax.ShapeDtypeStruct((NC, N, F), jnp.float32),
    )(x, W_in, b_in.reshape(1, 256), W1, degp)


def _pre_half(a_ref, dinv, b_ref, j):
    return dinv * a_ref[j] + b_ref[0, pl.ds(j * F, F)][None, :]


def _tc_stats_body(a_ref, degp_ref, b_ref, stat_ref):
    i = pl.program_id(0)
    dinv = _dinv(degp_ref)
    for j in range(2):
        pre = _pre_half(a_ref, dinv, b_ref, j)
        sm = jnp.sum(pre, axis=0, keepdims=True)
        sq = jnp.sum(pre * pre, axis=0, keepdims=True)

        @pl.when(i == 0)
        def _():
            stat_ref[0:1, pl.ds(j * F, F)] = sm
            stat_ref[1:2, pl.ds(j * F, F)] = sq

        @pl.when(i > 0)
        def _():
            stat_ref[0:1, pl.ds(j * F, F)] += sm
            stat_ref[1:2, pl.ds(j * F, F)] += sq


def _tc_stats(a, degp, b):
    return pl.pallas_call(
        _tc_stats_body,
        grid=(GRID,),
        in_specs=[
            pl.BlockSpec((NC, BR, F), lambda i: (0, i, 0)),
            pl.BlockSpec((BR, 128), lambda i: (i, 0)),
            _full((1, 256)),
        ],
        out_specs=_full((8, 256)),
        out_shape=jax.ShapeDtypeStruct((8, 256), jnp.float32),
    )(a, degp, b.reshape(1, 256))


def _norm_half(a_ref, b_ref, stat_ref, gamma_ref, beta_ref, j, dinv):
    pre = _pre_half(a_ref, dinv, b_ref, j)
    m = stat_ref[0:1, pl.ds(j * F, F)] * (1.0 / N)
    sq = stat_ref[1:2, pl.ds(j * F, F)] * (1.0 / N)
    v = sq - m * m
    return _gelu((pre - m) * lax.rsqrt(v + 1e-5)
                 * gamma_ref[0, pl.ds(j * F, F)][None, :]
                 + beta_ref[0, pl.ds(j * F, F)][None, :])


def _tc_mid_body(a_ref, degp_ref, b_ref, stat_ref, gamma_ref, beta_ref,
                 w_ref, g_ref):
    dinv = _dinv(degp_ref)
    for j in range(2):
        h_j = dinv * _norm_half(a_ref, b_ref, stat_ref, gamma_ref, beta_ref,
                                j, dinv)
        for q in range(2):
            blk = _mm(h_j, w_ref[pl.ds(j * F, F), pl.ds(q * F, F)])
            if j == 0:
                g_ref[q] = blk
            else:
                g_ref[q] += blk


def _tc_mid(a, degp, b, stat, gamma, beta, W):
    return pl.pallas_call(
        _tc_mid_body,
        grid=(GRID,),
        in_specs=[
            pl.BlockSpec((NC, BR, F), lambda i: (0, i, 0)),
            pl.BlockSpec((BR, 128), lambda i: (i, 0)),
            _full((1, 256)),
            _full((8, 256)),
            _full((1, 256)),
            _full((1, 256)),
            _full((256, 256)),
        ],
        out_specs=pl.BlockSpec((NC, BR, F), lambda i: (0, i, 0)),
        out_shape=jax.ShapeDtypeStruct((NC, N, F), jnp.float32),
    )(a, degp, b.reshape(1, 256), stat, gamma.reshape(1, 256),
      beta.reshape(1, 256), W)


def _tc_final_body(a_ref, degp_ref, b_ref, stat_ref, gamma_ref, beta_ref,
                   wo_ref, bo_ref, out_ref):
    dinv = _dinv(degp_ref)
    for j in range(2):
        h_j = _norm_half(a_ref, b_ref, stat_ref, gamma_ref, beta_ref, j,
                         dinv)
        blk = _mm(h_j, wo_ref[pl.ds(j * F, F), :])
        if j == 0:
            out_ref[...] = blk + bo_ref[...]
        else:
            out_ref[...] += blk


def _tc_final(a, degp, b, stat, gamma, beta, W_out, b_out):
    nout = W_out.shape[1]
    return pl.pallas_call(
        _tc_final_body,
        grid=(GRID,),
        in_specs=[
            pl.BlockSpec((NC, BR, F), lambda i: (0, i, 0)),
            pl.BlockSpec((BR, 128), lambda i: (i, 0)),
            _full((1, 256)),
            _full((8, 256)),
            _full((1, 256)),
            _full((1, 256)),
            _full((256, nout)),
            _full((1, nout)),
        ],
        out_specs=pl.BlockSpec((BR, nout), lambda i: (i, 0)),
        out_shape=jax.ShapeDtypeStruct((N, nout), jnp.float32),
    )(a, degp, b.reshape(1, 256), stat, gamma.reshape(1, 256),
      beta.reshape(1, 256), W_out, b_out.reshape(1, nout))


def kernel(x, edge_index, W_in, b_in, W1, b1, W2, b2, W3, b3, gamma, beta,
           W_out, b_out):
    src = edge_index[0]
    dst = edge_index[1]
    src2 = jnp.concatenate([src, src + jnp.int32(N)])

    ones_flat = jnp.ones((NC * N, F), jnp.float32)
    degp = _agg_call(ones_flat, src2, dst)

    g1 = _tc_pre(x, W_in, b_in, W1, degp)
    a1 = _agg_call(g1.reshape(NC * N, F), src2, dst).reshape(NC, N, F)
    s1 = _tc_stats(a1, degp, b1)
    g2 = _tc_mid(a1, degp, b1, s1, gamma, beta, W2)
    a2 = _agg_call(g2.reshape(NC * N, F), src2, dst).reshape(NC, N, F)
    s2 = _tc_stats(a2, degp, b2)
    g3 = _tc_mid(a2, degp, b2, s2, gamma, beta, W3)
    a3 = _agg_call(g3.reshape(NC * N, F), src2, dst).reshape(NC, N, F)
    s3 = _tc_stats(a3, degp, b3)
    return _tc_final(a3, degp, b3, s3, gamma, beta, W_out, b_out)


# speedup vs baseline: 16.7560x; 1.1625x over previous
import functools

import jax
import jax.numpy as jnp
from jax import lax
from jax.experimental import pallas as pl
from jax.experimental.pallas import tpu as pltpu
from jax.experimental.pallas import tpu_sc as plsc

N = 10000
E = 320000
F = 128
NC = 2
NS = 16
CH = 80

EPT_AGG = E // NS
NCH_AGG = EPT_AGG // CH
EPT_DEG = E // (NC * NS)
NCH_DEG = EPT_DEG // CH
STRIPE = 640
LAST = N - 15 * STRIPE

BR = 2000
GRID = N // BR


@functools.cache
def _mesh():
    return plsc.VectorSubcoreMesh(
        core_axis_name="c", subcore_axis_name="s", num_cores=NC,
        num_subcores=NS,
    )


def _stripe_copy(src_at, dst_at, s):
    @pl.when(s < NS - 1)
    def _():
        pltpu.sync_copy(src_at(s * STRIPE, STRIPE), dst_at(s * STRIPE, STRIPE))

    @pl.when(s == NS - 1)
    def _():
        pltpu.sync_copy(src_at(15 * STRIPE, LAST), dst_at(15 * STRIPE, LAST))


IBLK = 4000
NB_BLK = EPT_AGG // IBLK
CH_BLK = IBLK // CH
STEPS = CH_BLK // 2


def _agg_body(g_hbm, src2_hbm, dst_hbm, out_hbm, iall_s, iall_d, isrc_a,
              idst_a, isrc_b, idst_b, rows_a, rows_b, acc_sh, sem_a, sem_b):
    c = lax.axis_index("c")
    s = lax.axis_index("s")
    _stripe_copy(lambda o, n: g_hbm.at[pl.ds(c * N + o, n)],
                 lambda o, n: acc_sh.at[pl.ds(o, n)], s)
    plsc.subcore_barrier()

    def fill(iv_s, iv_d, k):
        for m in range(CH // 16):
            sl = pl.ds(pl.multiple_of(k * CH + m * 16, 16), 16)
            iv_s[pl.ds(m * 16, 16)] = iall_s[sl]
            iv_d[pl.ds(m * 16, 16)] = iall_d[sl]

    def block(b, _):
        bbase = pl.multiple_of(s * EPT_AGG + b * IBLK, CH)
        pltpu.sync_copy(src2_hbm.at[pl.ds(c * E + bbase, IBLK)], iall_s)
        pltpu.sync_copy(dst_hbm.at[pl.ds(bbase, IBLK)], iall_d)
        fill(isrc_a, idst_a, 0)
        pltpu.async_copy(g_hbm.at[isrc_a], rows_a, sem_a)

        def step(t, _):
            e = 2 * t
            fill(isrc_b, idst_b, e + 1)
            pltpu.async_copy(g_hbm.at[isrc_b], rows_b, sem_b)
            pltpu.make_async_copy(g_hbm.at[isrc_a], rows_a, sem_a).wait()
            pltpu.sync_copy(rows_a, acc_sh.at[idst_a], add=True)

            @pl.when(t < STEPS - 1)
            def _():
                fill(isrc_a, idst_a, e + 2)
                pltpu.async_copy(g_hbm.at[isrc_a], rows_a, sem_a)

            pltpu.make_async_copy(g_hbm.at[isrc_b], rows_b, sem_b).wait()
            pltpu.sync_copy(rows_b, acc_sh.at[idst_b], add=True)
            return 0

        lax.fori_loop(0, STEPS, step, 0)
        return 0

    lax.fori_loop(0, NB_BLK, block, 0)
    plsc.subcore_barrier()
    _stripe_copy(lambda o, n: acc_sh.at[pl.ds(o, n)],
                 lambda o, n: out_hbm.at[pl.ds(c * N + o, n)], s)


@functools.cache
def _agg_kernel():
    return pl.kernel(
        _agg_body,
        out_type=jax.ShapeDtypeStruct((NC * N, F), jnp.float32),
        mesh=_mesh(),
        scratch_types=[
            pltpu.VMEM((IBLK,), jnp.int32),
            pltpu.VMEM((IBLK,), jnp.int32),
            pltpu.VMEM((CH,), jnp.int32),
            pltpu.VMEM((CH,), jnp.int32),
            pltpu.VMEM((CH,), jnp.int32),
            pltpu.VMEM((CH,), jnp.int32),
            pltpu.VMEM((CH, F), jnp.float32),
            pltpu.VMEM((CH, F), jnp.float32),
            pltpu.VMEM_SHARED((N, F), jnp.float32),
            pltpu.SemaphoreType.DMA,
            pltpu.SemaphoreType.DMA,
        ],
    )


IBLK_DEG = 2000
NB_DEG = EPT_DEG // IBLK_DEG
CHB_DEG = IBLK_DEG // CH


def _deg_body(dst_hbm, ones_hbm, zeros_hbm, out_hbm, iall_d, idst_v, ones_v,
              acc_sh):
    c = lax.axis_index("c")
    s = lax.axis_index("s")
    pltpu.sync_copy(ones_hbm, ones_v)
    _stripe_copy(lambda o, n: zeros_hbm.at[pl.ds(o, n)],
                 lambda o, n: acc_sh.at[pl.ds(o, n)], s)
    plsc.subcore_barrier()

    def fill(k):
        for m in range(CH // 16):
            sl = pl.ds(pl.multiple_of(k * CH + m * 16, 16), 16)
            idst_v[pl.ds(m * 16, 16)] = iall_d[sl]

    def block(b, _):
        bbase = pl.multiple_of((c * NS + s) * EPT_DEG + b * IBLK_DEG, CH)
        pltpu.sync_copy(dst_hbm.at[pl.ds(bbase, IBLK_DEG)], iall_d)

        def chunk(k, _):
            fill(k)
            pltpu.sync_copy(ones_v, acc_sh.at[idst_v], add=True)
            return 0

        lax.fori_loop(0, CHB_DEG, chunk, 0)
        return 0

    lax.fori_loop(0, NB_DEG, block, 0)
    plsc.subcore_barrier()
    _stripe_copy(lambda o, n: acc_sh.at[pl.ds(o, n)],
                 lambda o, n: out_hbm.at[pl.ds(c * N + o, n)], s)


@functools.cache
def _deg_kernel():
    return pl.kernel(
        _deg_body,
        out_type=jax.ShapeDtypeStruct((NC * N, F), jnp.float32),
        mesh=_mesh(),
        scratch_types=[
            pltpu.VMEM((IBLK_DEG,), jnp.int32),
            pltpu.VMEM((CH,), jnp.int32),
            pltpu.VMEM((CH, F), jnp.float32),
            pltpu.VMEM_SHARED((N, F), jnp.float32),
        ],
    )


def _deg_call(dst):
    ones = jnp.ones((CH, F), jnp.float32)
    zeros = jnp.zeros((N, F), jnp.float32)
    return _deg_kernel()(dst, ones, zeros).reshape(NC, N, F)


def _agg_call(g_flat, src2, dst):
    return _agg_kernel()(g_flat, src2, dst)


def _mm(a, b):
    return jnp.dot(a, b, precision=lax.Precision.HIGHEST,
                   preferred_element_type=jnp.float32)


def _gelu(v):
    return 0.5 * v * (1.0 + lax.erf(v * 0.7071067811865476))


def _dinv(degp_ref):
    return lax.rsqrt(degp_ref[0, :, 0:1] + degp_ref[1, :, 0:1] + 1.0)


def _full(shape):
    return pl.BlockSpec(shape, lambda i: (0,) * len(shape))


def _tc_pre_body(x_ref, wi_ref, bi_ref, w1_ref, degp_ref, g_ref):
    dinv = _dinv(degp_ref)
    x = x_ref[...]
    for j in range(2):
        h_j = dinv * _gelu(_mm(x, wi_ref[:, j * F:(j + 1) * F])
                           + bi_ref[0, pl.ds(j * F, F)][None, :])
        for q in range(2):
            blk = _mm(h_j, w1_ref[pl.ds(j * F, F), pl.ds(q * F, F)])
            if j == 0:
                g_ref[q] = blk
            else:
                g_ref[q] += blk


def _tc_pre(x, W_in, b_in, W1, degp):
    return pl.pallas_call(
        _tc_pre_body,
        grid=(GRID,),
        in_specs=[
            pl.BlockSpec((BR, 128), lambda i: (i, 0)),
            _full((128, 256)),
            _full((1, 256)),
            _full((256, 256)),
            pl.BlockSpec((NC, BR, F), lambda i: (0, i, 0)),
        ],
        out_specs=pl.BlockSpec((NC, BR, F), lambda i: (0, i, 0)),
        out_shape=jax.ShapeDtypeStruct((NC, N, F), jnp.float32),
    )(x, W_in, b_in.reshape(1, 256), W1, degp)


def _pre_half(a_ref, dinv, b_ref, j):
    return dinv * a_ref[j] + b_ref[0, pl.ds(j * F, F)][None, :]


def _tc_stats_body(a_ref, degp_ref, b_ref, stat_ref):
    i = pl.program_id(0)
    dinv = _dinv(degp_ref)
    for j in range(2):
        pre = _pre_half(a_ref, dinv, b_ref, j)
        sm = jnp.sum(pre, axis=0, keepdims=True)
        sq = jnp.sum(pre * pre, axis=0, keepdims=True)

        @pl.when(i == 0)
        def _():
            stat_ref[0:1, pl.ds(j * F, F)] = sm
            stat_ref[1:2, pl.ds(j * F, F)] = sq

        @pl.when(i > 0)
        def _():
            stat_ref[0:1, pl.ds(j * F, F)] += sm
            stat_ref[1:2, pl.ds(j * F, F)] += sq


def _tc_stats(a, degp, b):
    return pl.pallas_call(
        _tc_stats_body,
        grid=(GRID,),
        in_specs=[
            pl.BlockSpec((NC, BR, F), lambda i: (0, i, 0)),
            pl.BlockSpec((NC, BR, F), lambda i: (0, i, 0)),
            _full((1, 256)),
        ],
        out_specs=_full((8, 256)),
        out_shape=jax.ShapeDtypeStruct((8, 256), jnp.float32),
    )(a, degp, b.reshape(1, 256))


def _norm_half(a_ref, b_ref, stat_ref, gamma_ref, beta_ref, j, dinv):
    pre = _pre_half(a_ref, dinv, b_ref, j)
    m = stat_ref[0:1, pl.ds(j * F, F)] * (1.0 / N)
    sq = stat_ref[1:2, pl.ds(j * F, F)] * (1.0 / N)
    v = sq - m * m
    return _gelu((pre - m) * lax.rsqrt(v + 1e-5)
                 * gamma_ref[0, pl.ds(j * F, F)][None, :]
                 + beta_ref[0, pl.ds(j * F, F)][None, :])


def _tc_mid_body(a_ref, degp_ref, b_ref, stat_ref, gamma_ref, beta_ref,
                 w_ref, g_ref):
    dinv = _dinv(degp_ref)
    for j in range(2):
        h_j = dinv * _norm_half(a_ref, b_ref, stat_ref, gamma_ref, beta_ref,
                                j, dinv)
        for q in range(2):
            blk = _mm(h_j, w_ref[pl.ds(j * F, F), pl.ds(q * F, F)])
            if j == 0:
                g_ref[q] = blk
            else:
                g_ref[q] += blk


def _tc_mid(a, degp, b, stat, gamma, beta, W):
    return pl.pallas_call(
        _tc_mid_body,
        grid=(GRID,),
        in_specs=[
            pl.BlockSpec((NC, BR, F), lambda i: (0, i, 0)),
            pl.BlockSpec((NC, BR, F), lambda i: (0, i, 0)),
            _full((1, 256)),
            _full((8, 256)),
            _full((1, 256)),
            _full((1, 256)),
            _full((256, 256)),
        ],
        out_specs=pl.BlockSpec((NC, BR, F), lambda i: (0, i, 0)),
        out_shape=jax.ShapeDtypeStruct((NC, N, F), jnp.float32),
    )(a, degp, b.reshape(1, 256), stat, gamma.reshape(1, 256),
      beta.reshape(1, 256), W)


def _tc_final_body(a_ref, degp_ref, b_ref, stat_ref, gamma_ref, beta_ref,
                   wo_ref, bo_ref, out_ref):
    dinv = _dinv(degp_ref)
    for j in range(2):
        h_j = _norm_half(a_ref, b_ref, stat_ref, gamma_ref, beta_ref, j,
                         dinv)
        blk = _mm(h_j, wo_ref[pl.ds(j * F, F), :])
        if j == 0:
            out_ref[...] = blk + bo_ref[...]
        else:
            out_ref[...] += blk


def _tc_final(a, degp, b, stat, gamma, beta, W_out, b_out):
    nout = W_out.shape[1]
    return pl.pallas_call(
        _tc_final_body,
        grid=(GRID,),
        in_specs=[
            pl.BlockSpec((NC, BR, F), lambda i: (0, i, 0)),
            pl.BlockSpec((NC, BR, F), lambda i: (0, i, 0)),
            _full((1, 256)),
            _full((8, 256)),
            _full((1, 256)),
            _full((1, 256)),
            _full((256, nout)),
            _full((1, nout)),
        ],
        out_specs=pl.BlockSpec((BR, nout), lambda i: (i, 0)),
        out_shape=jax.ShapeDtypeStruct((N, nout), jnp.float32),
    )(a, degp, b.reshape(1, 256), stat, gamma.reshape(1, 256),
      beta.reshape(1, 256), W_out, b_out.reshape(1, nout))


def kernel(x, edge_index, W_in, b_in, W1, b1, W2, b2, W3, b3, gamma, beta,
           W_out, b_out):
    src = edge_index[0]
    dst = edge_index[1]
    src2 = jnp.concatenate([src, src + jnp.int32(N)])

    degp = _deg_call(dst)

    g1 = _tc_pre(x, W_in, b_in, W1, degp)
    a1 = _agg_call(g1.reshape(NC * N, F), src2, dst).reshape(NC, N, F)
    s1 = _tc_stats(a1, degp, b1)
    g2 = _tc_mid(a1, degp, b1, s1, gamma, beta, W2)
    a2 = _agg_call(g2.reshape(NC * N, F), src2, dst).reshape(NC, N, F)
    s2 = _tc_stats(a2, degp, b2)
    g3 = _tc_mid(a2, degp, b2, s2, gamma, beta, W3)
    a3 = _agg_call(g3.reshape(NC * N, F), src2, dst).reshape(NC, N, F)
    s3 = _tc_stats(a3, degp, b3)
    return _tc_final(a3, degp, b3, s3, gamma, beta, W_out, b_out)
